# baseline (device time: 199152 ns/iter reference)
import jax
import jax.numpy as jnp
from jax import lax
from jax.experimental import pallas as pl
from jax.experimental.pallas import tpu as pltpu

N_DEV = 4


def kernel(x, dest):
    m, n = x.shape
    d2 = dest.reshape(16, 128)
    dm, dn = d2.shape

    m2 = m // 2

    def body(x_ref, d_ref, xout_ref, dout_ref,
             send_xr, recv_xr, send_xl, recv_xl, send_d, recv_d):
        my_x = lax.axis_index("x")
        my_y = lax.axis_index("y")
        my_z = lax.axis_index("z")
        left = lax.rem(my_z - 1 + N_DEV, N_DEV)
        right = lax.rem(my_z + 1, N_DEV)

        barrier_sem = pltpu.get_barrier_semaphore()
        for nbr in [left, right]:
            pl.semaphore_signal(
                barrier_sem, inc=1,
                device_id=(my_x, my_y, nbr),
                device_id_type=pl.DeviceIdType.MESH,
            )
        pl.semaphore_wait(barrier_sem, 2)

        xout_ref[pl.ds(my_z * m, m), :] = x_ref[...].astype(jnp.bfloat16)
        dout_ref[pl.ds(my_z * dm, dm), :] = d_ref[...]

        for h in range(N_DEV - 1):
            origin_r = lax.rem(my_z - h + N_DEV, N_DEV)
            origin_l = lax.rem(my_z + h, N_DEV)
            rxr = pltpu.make_async_remote_copy(
                src_ref=xout_ref.at[pl.ds(origin_r * m, m2), :],
                dst_ref=xout_ref.at[pl.ds(origin_r * m, m2), :],
                send_sem=send_xr.at[h],
                recv_sem=recv_xr.at[h],
                device_id=(my_x, my_y, right),
                device_id_type=pl.DeviceIdType.MESH,
            )
            rxl = pltpu.make_async_remote_copy(
                src_ref=xout_ref.at[pl.ds(origin_l * m + m2, m2), :],
                dst_ref=xout_ref.at[pl.ds(origin_l * m + m2, m2), :],
                send_sem=send_xl.at[h],
                recv_sem=recv_xl.at[h],
                device_id=(my_x, my_y, left),
                device_id_type=pl.DeviceIdType.MESH,
            )
            rd = pltpu.make_async_remote_copy(
                src_ref=dout_ref.at[pl.ds(origin_r * dm, dm), :],
                dst_ref=dout_ref.at[pl.ds(origin_r * dm, dm), :],
                send_sem=send_d.at[h],
                recv_sem=recv_d.at[h],
                device_id=(my_x, my_y, right),
                device_id_type=pl.DeviceIdType.MESH,
            )
            rxr.start()
            rxl.start()
            rd.start()
            rxr.wait()
            rxl.wait()
            rd.wait()

    x_full, d_full = pl.pallas_call(
        body,
        out_shape=(
            jax.ShapeDtypeStruct((N_DEV * m, n), jnp.bfloat16),
            jax.ShapeDtypeStruct((N_DEV * dm, dn), jnp.int32),
        ),
        in_specs=[
            pl.BlockSpec(memory_space=pltpu.VMEM),
            pl.BlockSpec(memory_space=pltpu.VMEM),
        ],
        out_specs=(
            pl.BlockSpec(memory_space=pltpu.VMEM),
            pl.BlockSpec(memory_space=pltpu.VMEM),
        ),
        scratch_shapes=[
            pltpu.SemaphoreType.DMA((N_DEV - 1,)),
            pltpu.SemaphoreType.DMA((N_DEV - 1,)),
            pltpu.SemaphoreType.DMA((N_DEV - 1,)),
            pltpu.SemaphoreType.DMA((N_DEV - 1,)),
            pltpu.SemaphoreType.DMA((N_DEV - 1,)),
            pltpu.SemaphoreType.DMA((N_DEV - 1,)),
        ],
        compiler_params=pltpu.CompilerParams(collective_id=0),
    )(x, d2)

    dest_full = d_full.reshape(N_DEV * m)
    order = jnp.argsort(dest_full, stable=True)
    my_z = lax.axis_index("z")
    idx = lax.dynamic_slice(order, (my_z * m,), (m,))
    return jnp.take(x_full, idx, axis=0)


# device time: 77309 ns/iter; 2.5761x vs baseline; 2.5761x over previous
import functools

import jax
import jax.numpy as jnp
from jax import lax
from jax.experimental import pallas as pl
from jax.experimental.pallas import tpu as pltpu

N_DEV = 4


def _counts_gather(cnt_row):

    def body(c_ref, out_ref, send_sems, recv_sems):
        my_x = lax.axis_index("x")
        my_y = lax.axis_index("y")
        my_z = lax.axis_index("z")

        barrier_sem = pltpu.get_barrier_semaphore()
        for k in range(1, N_DEV):
            pl.semaphore_signal(
                barrier_sem, inc=1,
                device_id=(my_x, my_y, lax.rem(my_z + k, N_DEV)),
                device_id_type=pl.DeviceIdType.MESH,
            )
        pl.semaphore_wait(barrier_sem, N_DEV - 1)

        out_ref[pl.ds(my_z, 1), :] = c_ref[...]

        sends = []
        for k in range(1, N_DEV):
            peer = lax.rem(my_z + k, N_DEV)
            rdma = pltpu.make_async_remote_copy(
                src_ref=out_ref.at[pl.ds(my_z, 1), :],
                dst_ref=out_ref.at[pl.ds(my_z, 1), :],
                send_sem=send_sems.at[k - 1],
                recv_sem=recv_sems.at[k - 1],
                device_id=(my_x, my_y, peer),
                device_id_type=pl.DeviceIdType.MESH,
            )
            rdma.start()
            sends.append(rdma)
        for k in range(1, N_DEV):
            src_z = lax.rem(my_z - k + N_DEV, N_DEV)
            recv = pltpu.make_async_remote_copy(
                src_ref=out_ref.at[pl.ds(src_z, 1), :],
                dst_ref=out_ref.at[pl.ds(src_z, 1), :],
                send_sem=send_sems.at[k - 1],
                recv_sem=recv_sems.at[k - 1],
                device_id=(my_x, my_y, src_z),
                device_id_type=pl.DeviceIdType.MESH,
            )
            recv.wait_recv()
        for rdma in sends:
            rdma.wait_send()

    return pl.pallas_call(
        body,
        out_shape=jax.ShapeDtypeStruct((N_DEV, 128), jnp.int32),
        in_specs=[pl.BlockSpec(memory_space=pltpu.VMEM)],
        out_specs=pl.BlockSpec(memory_space=pltpu.VMEM),
        scratch_shapes=[
            pltpu.SemaphoreType.DMA((N_DEV - 1,)),
            pltpu.SemaphoreType.DMA((N_DEV - 1,)),
        ],
        compiler_params=pltpu.CompilerParams(collective_id=0),
    )(cnt_row)


def _row_scatter(xb, tgt, m, n):

    def body(xb_ref, t_ref, out_ref, send_sem, recv_sem):
        my_x = lax.axis_index("x")
        my_y = lax.axis_index("y")
        my_z = lax.axis_index("z")

        barrier_sem = pltpu.get_barrier_semaphore()
        for k in range(1, N_DEV):
            pl.semaphore_signal(
                barrier_sem, inc=1,
                device_id=(my_x, my_y, lax.rem(my_z + k, N_DEV)),
                device_id_type=pl.DeviceIdType.MESH,
            )
        pl.semaphore_wait(barrier_sem, N_DEV - 1)

        def loop(j, carry):
            g = t_ref[j]
            d = g // m
            o = g - d * m
            copy = pltpu.make_async_remote_copy(
                src_ref=xb_ref.at[pl.ds(j, 1)],
                dst_ref=out_ref.at[pl.ds(o, 1)],
                send_sem=send_sem,
                recv_sem=recv_sem,
                device_id=(my_x, my_y, d),
                device_id_type=pl.DeviceIdType.MESH,
            )
            copy.start()
            return carry

        W = 128
        n_waves = m // W
        for w in range(n_waves):
            if w >= 2:
                drain = pltpu.make_async_remote_copy(
                    src_ref=xb_ref.at[pl.ds((w - 2) * W, W)],
                    dst_ref=out_ref.at[pl.ds(0, W)],
                    send_sem=send_sem,
                    recv_sem=recv_sem,
                    device_id=(my_x, my_y, my_z),
                    device_id_type=pl.DeviceIdType.MESH,
                )
                drain.wait_send()
            lax.fori_loop(w * W, (w + 1) * W, loop, 0)

        tail = pltpu.make_async_remote_copy(
            src_ref=xb_ref.at[pl.ds((n_waves - 2) * W, 2 * W)],
            dst_ref=out_ref.at[pl.ds(0, 2 * W)],
            send_sem=send_sem,
            recv_sem=recv_sem,
            device_id=(my_x, my_y, my_z),
            device_id_type=pl.DeviceIdType.MESH,
        )
        tail.wait_send()
        done = pltpu.make_async_remote_copy(
            src_ref=xb_ref,
            dst_ref=out_ref,
            send_sem=send_sem,
            recv_sem=recv_sem,
            device_id=(my_x, my_y, my_z),
            device_id_type=pl.DeviceIdType.MESH,
        )
        done.wait_recv()

        @functools.partial(
            pl.run_scoped, second_barrier=pltpu.SemaphoreType.REGULAR
        )
        def _(second_barrier):
            for k in range(1, N_DEV):
                pl.semaphore_signal(
                    second_barrier, inc=1,
                    device_id=(my_x, my_y, lax.rem(my_z + k, N_DEV)),
                    device_id_type=pl.DeviceIdType.MESH,
                )
            pl.semaphore_wait(second_barrier, N_DEV - 1)

    return pl.pallas_call(
        body,
        out_shape=jax.ShapeDtypeStruct((m, n // 128, 128), jnp.bfloat16),
        in_specs=[
            pl.BlockSpec(memory_space=pltpu.VMEM),
            pl.BlockSpec(memory_space=pltpu.SMEM),
        ],
        out_specs=pl.BlockSpec(memory_space=pltpu.VMEM),
        scratch_shapes=[
            pltpu.SemaphoreType.DMA,
            pltpu.SemaphoreType.DMA,
        ],
        compiler_params=pltpu.CompilerParams(collective_id=1),
    )(xb, tgt)


def kernel(x, dest):
    m, n = x.shape
    my_z = lax.axis_index("z")

    onehot = (dest[:, None] == jnp.arange(N_DEV)[None, :]).astype(jnp.int32)
    counts = onehot.sum(axis=0)
    cnt_row = jnp.zeros((1, 128), jnp.int32).at[0, :N_DEV].set(counts)

    C = _counts_gather(cnt_row)[:, :N_DEV]

    excl = jnp.cumsum(C, axis=0) - C
    bases = lax.dynamic_slice(excl, (my_z, 0), (1, N_DEV))[0]
    prefix = jnp.cumsum(onehot, axis=0) - onehot
    o_local = jnp.take_along_axis(prefix, dest[:, None], axis=1)[:, 0]
    tgt = dest * m + jnp.take(bases, dest) + o_local

    xb = x.astype(jnp.bfloat16).reshape(m, n // 128, 128)
    return _row_scatter(xb, tgt, m, n).reshape(m, n)


# device time: 67489 ns/iter; 2.9509x vs baseline; 1.1455x over previous
import functools

import jax
import jax.numpy as jnp
from jax import lax
from jax.experimental import pallas as pl
from jax.experimental.pallas import tpu as pltpu

N_DEV = 4


def _counts_gather(d2):

    def body(d_ref, out_ref, send_sems, recv_sems):
        my_x = lax.axis_index("x")
        my_y = lax.axis_index("y")
        my_z = lax.axis_index("z")

        barrier_sem = pltpu.get_barrier_semaphore()
        for k in range(1, N_DEV):
            pl.semaphore_signal(
                barrier_sem, inc=1,
                device_id=(my_x, my_y, lax.rem(my_z + k, N_DEV)),
                device_id_type=pl.DeviceIdType.MESH,
            )
        pl.semaphore_wait(barrier_sem, N_DEV - 1)

        lane = lax.broadcasted_iota(jnp.int32, (1, 128), 1)
        row = jnp.zeros((1, 128), jnp.int32)
        dv = d_ref[...]
        for d in range(N_DEV):
            cnt = jnp.sum((dv == d).astype(jnp.int32))
            row = jnp.where(lane == d, cnt, row)
        out_ref[pl.ds(my_z, 1), :] = row

        sends = []
        for k in range(1, N_DEV):
            peer = lax.rem(my_z + k, N_DEV)
            rdma = pltpu.make_async_remote_copy(
                src_ref=out_ref.at[pl.ds(my_z, 1), :],
                dst_ref=out_ref.at[pl.ds(my_z, 1), :],
                send_sem=send_sems.at[k - 1],
                recv_sem=recv_sems.at[k - 1],
                device_id=(my_x, my_y, peer),
                device_id_type=pl.DeviceIdType.MESH,
            )
            rdma.start()
            sends.append(rdma)
        for k in range(1, N_DEV):
            src_z = lax.rem(my_z - k + N_DEV, N_DEV)
            recv = pltpu.make_async_remote_copy(
                src_ref=out_ref.at[pl.ds(src_z, 1), :],
                dst_ref=out_ref.at[pl.ds(src_z, 1), :],
                send_sem=send_sems.at[k - 1],
                recv_sem=recv_sems.at[k - 1],
                device_id=(my_x, my_y, src_z),
                device_id_type=pl.DeviceIdType.MESH,
            )
            recv.wait_recv()
        for rdma in sends:
            rdma.wait_send()

    return pl.pallas_call(
        body,
        out_shape=jax.ShapeDtypeStruct((N_DEV, 128), jnp.int32),
        in_specs=[pl.BlockSpec(memory_space=pltpu.VMEM)],
        out_specs=pl.BlockSpec(memory_space=pltpu.VMEM),
        scratch_shapes=[
            pltpu.SemaphoreType.DMA((N_DEV - 1,)),
            pltpu.SemaphoreType.DMA((N_DEV - 1,)),
        ],
        compiler_params=pltpu.CompilerParams(collective_id=0),
    )(d2)


def _row_scatter(x3, dest, bases, m, n):

    def body(x_ref, dest_ref, bases_ref, out_ref, xb_ref, cnt_ref,
             send_sem, recv_sem):
        my_x = lax.axis_index("x")
        my_y = lax.axis_index("y")
        my_z = lax.axis_index("z")

        barrier_sem = pltpu.get_barrier_semaphore()
        for k in range(1, N_DEV):
            pl.semaphore_signal(
                barrier_sem, inc=1,
                device_id=(my_x, my_y, lax.rem(my_z + k, N_DEV)),
                device_id_type=pl.DeviceIdType.MESH,
            )
        pl.semaphore_wait(barrier_sem, N_DEV - 1)

        xb_ref[...] = x_ref[...].astype(jnp.bfloat16)
        for d in range(N_DEV):
            cnt_ref[d] = 0

        def loop(j, carry):
            d = dest_ref[j]
            o = bases_ref[d] + cnt_ref[d]
            cnt_ref[d] = cnt_ref[d] + 1
            copy = pltpu.make_async_remote_copy(
                src_ref=xb_ref.at[pl.ds(j, 1)],
                dst_ref=out_ref.at[pl.ds(o, 1)],
                send_sem=send_sem,
                recv_sem=recv_sem,
                device_id=(my_x, my_y, d),
                device_id_type=pl.DeviceIdType.MESH,
            )
            copy.start()
            return carry

        W = 128
        n_waves = m // W
        for w in range(n_waves):
            if w >= 2:
                drain = pltpu.make_async_remote_copy(
                    src_ref=xb_ref.at[pl.ds((w - 2) * W, W)],
                    dst_ref=out_ref.at[pl.ds(0, W)],
                    send_sem=send_sem,
                    recv_sem=recv_sem,
                    device_id=(my_x, my_y, my_z),
                    device_id_type=pl.DeviceIdType.MESH,
                )
                drain.wait_send()
            lax.fori_loop(w * W, (w + 1) * W, loop, 0)

        tail = pltpu.make_async_remote_copy(
            src_ref=xb_ref.at[pl.ds((n_waves - 2) * W, 2 * W)],
            dst_ref=out_ref.at[pl.ds(0, 2 * W)],
            send_sem=send_sem,
            recv_sem=recv_sem,
            device_id=(my_x, my_y, my_z),
            device_id_type=pl.DeviceIdType.MESH,
        )
        tail.wait_send()
        done = pltpu.make_async_remote_copy(
            src_ref=xb_ref,
            dst_ref=out_ref,
            send_sem=send_sem,
            recv_sem=recv_sem,
            device_id=(my_x, my_y, my_z),
            device_id_type=pl.DeviceIdType.MESH,
        )
        done.wait_recv()

        @functools.partial(
            pl.run_scoped, second_barrier=pltpu.SemaphoreType.REGULAR
        )
        def _(second_barrier):
            for k in range(1, N_DEV):
                pl.semaphore_signal(
                    second_barrier, inc=1,
                    device_id=(my_x, my_y, lax.rem(my_z + k, N_DEV)),
                    device_id_type=pl.DeviceIdType.MESH,
                )
            pl.semaphore_wait(second_barrier, N_DEV - 1)

    return pl.pallas_call(
        body,
        out_shape=jax.ShapeDtypeStruct((m, n // 128, 128), jnp.bfloat16),
        in_specs=[
            pl.BlockSpec(memory_space=pltpu.VMEM),
            pl.BlockSpec(memory_space=pltpu.SMEM),
            pl.BlockSpec(memory_space=pltpu.SMEM),
        ],
        out_specs=pl.BlockSpec(memory_space=pltpu.VMEM),
        scratch_shapes=[
            pltpu.VMEM((m, n // 128, 128), jnp.bfloat16),
            pltpu.SMEM((N_DEV,), jnp.int32),
            pltpu.SemaphoreType.DMA,
            pltpu.SemaphoreType.DMA,
        ],
        compiler_params=pltpu.CompilerParams(collective_id=1),
    )(x3, dest, bases)


def kernel(x, dest):
    m, n = x.shape
    my_z = lax.axis_index("z")

    d2 = dest.reshape(16, 128)
    C = _counts_gather(d2)[:, :N_DEV]

    excl = jnp.cumsum(C, axis=0) - C
    bases = lax.dynamic_slice(excl, (my_z, 0), (1, N_DEV))[0]

    x3 = x.reshape(m, n // 128, 128)
    return _row_scatter(x3, dest, bases, m, n).reshape(m, n)


# device time: 67248 ns/iter; 2.9615x vs baseline; 1.0036x over previous
import functools

import jax
import jax.numpy as jnp
from jax import lax
from jax.experimental import pallas as pl
from jax.experimental.pallas import tpu as pltpu

N_DEV = 4


def _counts_gather(d2):

    def body(d_ref, out_ref, send_sems, recv_sems):
        my_x = lax.axis_index("x")
        my_y = lax.axis_index("y")
        my_z = lax.axis_index("z")

        barrier_sem = pltpu.get_barrier_semaphore()
        for k in range(1, N_DEV):
            pl.semaphore_signal(
                barrier_sem, inc=1,
                device_id=(my_x, my_y, lax.rem(my_z + k, N_DEV)),
                device_id_type=pl.DeviceIdType.MESH,
            )
        pl.semaphore_wait(barrier_sem, N_DEV - 1)

        lane = lax.broadcasted_iota(jnp.int32, (1, 128), 1)
        row = jnp.zeros((1, 128), jnp.int32)
        dv = d_ref[...]
        for d in range(N_DEV):
            cnt = jnp.sum((dv == d).astype(jnp.int32))
            row = jnp.where(lane == d, cnt, row)
        out_ref[pl.ds(my_z, 1), :] = row

        sends = []
        for k in range(1, N_DEV):
            peer = lax.rem(my_z + k, N_DEV)
            rdma = pltpu.make_async_remote_copy(
                src_ref=out_ref.at[pl.ds(my_z, 1), :],
                dst_ref=out_ref.at[pl.ds(my_z, 1), :],
                send_sem=send_sems.at[k - 1],
                recv_sem=recv_sems.at[k - 1],
                device_id=(my_x, my_y, peer),
                device_id_type=pl.DeviceIdType.MESH,
            )
            rdma.start()
            sends.append(rdma)
        for k in range(1, N_DEV):
            src_z = lax.rem(my_z - k + N_DEV, N_DEV)
            recv = pltpu.make_async_remote_copy(
                src_ref=out_ref.at[pl.ds(src_z, 1), :],
                dst_ref=out_ref.at[pl.ds(src_z, 1), :],
                send_sem=send_sems.at[k - 1],
                recv_sem=recv_sems.at[k - 1],
                device_id=(my_x, my_y, src_z),
                device_id_type=pl.DeviceIdType.MESH,
            )
            recv.wait_recv()
        for rdma in sends:
            rdma.wait_send()

    return pl.pallas_call(
        body,
        out_shape=jax.ShapeDtypeStruct((N_DEV, 128), jnp.int32),
        in_specs=[pl.BlockSpec(memory_space=pltpu.VMEM)],
        out_specs=pl.BlockSpec(memory_space=pltpu.VMEM),
        scratch_shapes=[
            pltpu.SemaphoreType.DMA((N_DEV - 1,)),
            pltpu.SemaphoreType.DMA((N_DEV - 1,)),
        ],
        compiler_params=pltpu.CompilerParams(collective_id=0),
    )(d2)


def _row_scatter(x3, dest, bases, m, n):

    def body(x_ref, dest_ref, bases_ref, out_ref, xb_ref, cnt_ref,
             send_sem, recv_sem):
        my_x = lax.axis_index("x")
        my_y = lax.axis_index("y")
        my_z = lax.axis_index("z")

        barrier_sem = pltpu.get_barrier_semaphore()
        for k in range(1, N_DEV):
            pl.semaphore_signal(
                barrier_sem, inc=1,
                device_id=(my_x, my_y, lax.rem(my_z + k, N_DEV)),
                device_id_type=pl.DeviceIdType.MESH,
            )
        pl.semaphore_wait(barrier_sem, N_DEV - 1)

        xb_ref[...] = x_ref[...].astype(jnp.bfloat16)
        for d in range(N_DEV):
            cnt_ref[d] = 0

        def loop(j, carry):
            d = dest_ref[j]
            o = bases_ref[d] + cnt_ref[d]
            cnt_ref[d] = cnt_ref[d] + 1
            copy = pltpu.make_async_remote_copy(
                src_ref=xb_ref.at[pl.ds(j, 1)],
                dst_ref=out_ref.at[pl.ds(o, 1)],
                send_sem=send_sem,
                recv_sem=recv_sem,
                device_id=(my_x, my_y, d),
                device_id_type=pl.DeviceIdType.MESH,
            )
            copy.start()
            return carry

        W = 256
        n_waves = m // W
        for w in range(n_waves):
            if w >= 2:
                drain = pltpu.make_async_remote_copy(
                    src_ref=xb_ref.at[pl.ds((w - 2) * W, W)],
                    dst_ref=out_ref.at[pl.ds(0, W)],
                    send_sem=send_sem,
                    recv_sem=recv_sem,
                    device_id=(my_x, my_y, my_z),
                    device_id_type=pl.DeviceIdType.MESH,
                )
                drain.wait_send()
            lax.fori_loop(w * W, (w + 1) * W, loop, 0, unroll=4)

        tail = pltpu.make_async_remote_copy(
            src_ref=xb_ref.at[pl.ds((n_waves - 2) * W, 2 * W)],
            dst_ref=out_ref.at[pl.ds(0, 2 * W)],
            send_sem=send_sem,
            recv_sem=recv_sem,
            device_id=(my_x, my_y, my_z),
            device_id_type=pl.DeviceIdType.MESH,
        )
        tail.wait_send()
        done = pltpu.make_async_remote_copy(
            src_ref=xb_ref,
            dst_ref=out_ref,
            send_sem=send_sem,
            recv_sem=recv_sem,
            device_id=(my_x, my_y, my_z),
            device_id_type=pl.DeviceIdType.MESH,
        )
        done.wait_recv()

        @functools.partial(
            pl.run_scoped, second_barrier=pltpu.SemaphoreType.REGULAR
        )
        def _(second_barrier):
            for k in range(1, N_DEV):
                pl.semaphore_signal(
                    second_barrier, inc=1,
                    device_id=(my_x, my_y, lax.rem(my_z + k, N_DEV)),
                    device_id_type=pl.DeviceIdType.MESH,
                )
            pl.semaphore_wait(second_barrier, N_DEV - 1)

    return pl.pallas_call(
        body,
        out_shape=jax.ShapeDtypeStruct((m, n // 128, 128), jnp.bfloat16),
        in_specs=[
            pl.BlockSpec(memory_space=pltpu.VMEM),
            pl.BlockSpec(memory_space=pltpu.SMEM),
            pl.BlockSpec(memory_space=pltpu.SMEM),
        ],
        out_specs=pl.BlockSpec(memory_space=pltpu.VMEM),
        scratch_shapes=[
            pltpu.VMEM((m, n // 128, 128), jnp.bfloat16),
            pltpu.SMEM((N_DEV,), jnp.int32),
            pltpu.SemaphoreType.DMA,
            pltpu.SemaphoreType.DMA,
        ],
        compiler_params=pltpu.CompilerParams(collective_id=1),
    )(x3, dest, bases)


def kernel(x, dest):
    m, n = x.shape
    my_z = lax.axis_index("z")

    d2 = dest.reshape(16, 128)
    C = _counts_gather(d2)[:, :N_DEV]

    excl = jnp.cumsum(C, axis=0) - C
    bases = lax.dynamic_slice(excl, (my_z, 0), (1, N_DEV))[0]

    x3 = x.reshape(m, n // 128, 128)
    return _row_scatter(x3, dest, bases, m, n).reshape(m, n)
